# baseline (speedup 1.0000x reference)
import functools

import jax
import jax.numpy as jnp
from jax import lax
from jax.experimental import pallas as pl
from jax.experimental.pallas import tpu as pltpu
from jax.experimental.pallas import tpu_sc as plsc

B = 16384
V = 32768
NB = 15
NC = 2
NS = 16
NW = NC * NS
BPW = B // NW
CH = 16
NCHUNK = BPW // CH
BLK = 128
NBLK = BPW // BLK
D = 512


def _pack_body(x_ref, lin_ref):
    i = lax.broadcasted_iota(jnp.int32, (NB, 1), 0)
    w = jnp.left_shift(jnp.int32(1), NB - 1 - i)
    lin_ref[...] = jnp.sum(x_ref[...] * w, axis=0)


_pack = pl.pallas_call(
    _pack_body,
    out_shape=jax.ShapeDtypeStruct((B,), jnp.int32),
)


def _gbody(params_hbm, lin_hbm, out_hbm, idx_v, stg0, stg1, asm,
           gsem0, gsem1, osem):
    wid = lax.axis_index("s") * NC + lax.axis_index("c")
    base = wid * BPW

    pltpu.sync_copy(lin_hbm.at[pl.ds(base, BPW)], idx_v)

    stgs = (stg0, stg1)
    gsems = (gsem0, gsem1)
    ivec = lax.broadcasted_iota(jnp.int32, (16,), 0)

    def fire(ci, b):
        v = idx_v[pl.ds(ci * CH, CH)]
        for l in range(CH):
            pltpu.async_copy(params_hbm.at[v[l]], stgs[b].at[l], gsems[b])

    def drain(b):
        pltpu.make_async_copy(params_hbm.at[pl.ds(0, CH)], stgs[b],
                              gsems[b]).wait()

    def transpose(b, col):
        def sbody(s, _):
            svec = jnp.full((16,), s, jnp.int32)
            q0 = s * 64
            for l in range(64):
                lvec = jnp.full((16,), l, jnp.int32)
                vals = plsc.load_gather(stgs[b], [ivec, svec, lvec])
                asm[q0 + l, pl.ds(col, 16)] = vals
            return 0
        lax.fori_loop(0, 8, sbody, 0)

    def start_out(g):
        pltpu.async_copy(asm, out_hbm.at[:, pl.ds(base + g * BLK, BLK)],
                         osem)

    def wait_out(g):
        pltpu.make_async_copy(asm,
                              out_hbm.at[:, pl.ds(base + g * BLK, BLK)],
                              osem).wait()

    fire(0, 0)

    def pair(j, _):
        c0 = 2 * j
        fire(c0 + 1, 1)
        drain(0)

        @pl.when(jnp.logical_and(j % 4 == 0, j >= 4))
        def _():
            wait_out(j // 4 - 1)

        transpose(0, (c0 % 8) * CH)

        @pl.when(j < NCHUNK // 2 - 1)
        def _():
            fire(c0 + 2, 0)

        drain(1)
        transpose(1, ((c0 + 1) % 8) * CH)

        @pl.when(j % 4 == 3)
        def _():
            start_out(j // 4)

        return 0

    lax.fori_loop(0, NCHUNK // 2, pair, 0)
    wait_out(NBLK - 1)


_gather = functools.partial(
    pl.kernel,
    mesh=plsc.VectorSubcoreMesh(core_axis_name="c", subcore_axis_name="s"),
    out_type=jax.ShapeDtypeStruct((D, B), jnp.float32),
    scratch_types=[
        pltpu.VMEM((BPW,), jnp.int32),
        pltpu.VMEM((CH, 8, 64), jnp.float32),
        pltpu.VMEM((CH, 8, 64), jnp.float32),
        pltpu.VMEM((D, BLK), jnp.float32),
        pltpu.SemaphoreType.DMA,
        pltpu.SemaphoreType.DMA,
        pltpu.SemaphoreType.DMA,
    ],
    compiler_params=pltpu.CompilerParams(use_tc_tiling_on_sc=True,
                                         needs_layout_passes=False),
)(_gbody)


def kernel(x, params):
    lin = _pack(x)
    table = params.reshape(V, 8, 64)
    out2d = _gather(table, lin)
    return out2d.reshape(8, 64, B).transpose(2, 0, 1)


# speedup vs baseline: 1.6462x; 1.6462x over previous
import functools

import jax
import jax.numpy as jnp
from jax import lax
from jax.experimental import pallas as pl
from jax.experimental.pallas import tpu as pltpu
from jax.experimental.pallas import tpu_sc as plsc

B = 16384
V = 32768
NB = 15
NC = 2
NS = 16
NW = NC * NS
BPW = B // NW
CH = 16
NCHUNK = BPW // CH
BLK = 128
NBLK = BPW // BLK
D = 512


def _pack_body(x_ref, lin_ref):
    i = lax.broadcasted_iota(jnp.int32, (NB, 1), 0)
    w = jnp.left_shift(jnp.int32(1), NB - 1 - i)
    lin_ref[...] = jnp.sum(x_ref[...] * w, axis=0)


_pack = pl.pallas_call(
    _pack_body,
    out_shape=jax.ShapeDtypeStruct((B,), jnp.int32),
)


def _gbody(params_hbm, lin_hbm, out_hbm, idx_v, stg0, stg1, asm,
           gsem0, gsem1, osem):
    wid = lax.axis_index("s") * NC + lax.axis_index("c")
    base = wid * BPW

    pltpu.sync_copy(lin_hbm.at[pl.ds(base, BPW)], idx_v)

    stgs = (stg0, stg1)
    gsems = (gsem0, gsem1)
    ivec = lax.broadcasted_iota(jnp.int32, (16,), 0)

    def fire(ci, b):
        v = idx_v[pl.ds(ci * CH, CH)]
        for l in range(CH):
            pltpu.async_copy(params_hbm.at[v[l]], stgs[b].at[l], gsems[b])

    def drain(b):
        pltpu.make_async_copy(params_hbm.at[pl.ds(0, CH)], stgs[b],
                              gsems[b]).wait()

    def transpose(b, col):
        @plsc.parallel_loop(0, D, step=1, unroll=8)
        def _(q):
            svec = jnp.full((16,), q // 64, jnp.int32)
            lvec = jnp.full((16,), q % 64, jnp.int32)
            vals = plsc.load_gather(stgs[b], [ivec, svec, lvec])
            asm[q, pl.ds(col, 16)] = vals

    def start_out(g):
        pltpu.async_copy(asm, out_hbm.at[:, pl.ds(base + g * BLK, BLK)],
                         osem)

    def wait_out(g):
        pltpu.make_async_copy(asm,
                              out_hbm.at[:, pl.ds(base + g * BLK, BLK)],
                              osem).wait()

    fire(0, 0)

    def pair(j, _):
        c0 = 2 * j
        fire(c0 + 1, 1)
        drain(0)

        @pl.when(jnp.logical_and(j % 4 == 0, j >= 4))
        def _():
            wait_out(j // 4 - 1)

        transpose(0, (c0 % 8) * CH)

        @pl.when(j < NCHUNK // 2 - 1)
        def _():
            fire(c0 + 2, 0)

        drain(1)
        transpose(1, ((c0 + 1) % 8) * CH)

        @pl.when(j % 4 == 3)
        def _():
            start_out(j // 4)

        return 0

    lax.fori_loop(0, NCHUNK // 2, pair, 0)
    wait_out(NBLK - 1)


_gather = functools.partial(
    pl.kernel,
    mesh=plsc.VectorSubcoreMesh(core_axis_name="c", subcore_axis_name="s"),
    out_type=jax.ShapeDtypeStruct((D, B), jnp.float32),
    scratch_types=[
        pltpu.VMEM((BPW,), jnp.int32),
        pltpu.VMEM((CH, 8, 64), jnp.float32),
        pltpu.VMEM((CH, 8, 64), jnp.float32),
        pltpu.VMEM((D, BLK), jnp.float32),
        pltpu.SemaphoreType.DMA,
        pltpu.SemaphoreType.DMA,
        pltpu.SemaphoreType.DMA,
    ],
    compiler_params=pltpu.CompilerParams(use_tc_tiling_on_sc=True,
                                         needs_layout_passes=False),
)(_gbody)


def kernel(x, params):
    lin = _pack(x)
    table = params.reshape(V, 8, 64)
    out2d = _gather(table, lin)
    return out2d.reshape(8, 64, B).transpose(2, 0, 1)


# speedup vs baseline: 1.6872x; 1.0249x over previous
import functools

import jax
import jax.numpy as jnp
from jax import lax
from jax.experimental import pallas as pl
from jax.experimental.pallas import tpu as pltpu
from jax.experimental.pallas import tpu_sc as plsc

B = 16384
V = 32768
NB = 15
NC = 2
NS = 16
NW = NC * NS
BPW = B // NW
CH = 16
NCHUNK = BPW // CH
BLK = 128
NBLK = BPW // BLK
D = 512


def _pack_body(x_ref, lin_ref):
    i = lax.broadcasted_iota(jnp.int32, (NB, 1), 0)
    w = jnp.left_shift(jnp.int32(1), NB - 1 - i)
    lin_ref[...] = jnp.sum(x_ref[...] * w, axis=0)


_pack = pl.pallas_call(
    _pack_body,
    out_shape=jax.ShapeDtypeStruct((B,), jnp.int32),
)


def _gbody(params_hbm, lin_hbm, out_hbm, idx_v, stg0, stg1, asm,
           gsem0, gsem1, osem):
    wid = lax.axis_index("s") * NC + lax.axis_index("c")
    base = wid * BPW

    pltpu.sync_copy(lin_hbm.at[pl.ds(base, BPW)], idx_v)

    stgs = (stg0, stg1)
    gsems = (gsem0, gsem1)
    ivec = lax.broadcasted_iota(jnp.int32, (16,), 0)

    def fire(ci, b):
        v = idx_v[pl.ds(ci * CH, CH)]
        for l in range(CH):
            pltpu.async_copy(params_hbm.at[v[l]], stgs[b].at[l], gsems[b])

    def drain(b):
        pltpu.make_async_copy(params_hbm.at[pl.ds(0, CH)], stgs[b],
                              gsems[b]).wait()

    def transpose(b, col):
        def souter(s, _):
            svec = jnp.full((16,), s, jnp.int32)
            q0 = s * 64

            @plsc.parallel_loop(0, 64, step=1, unroll=16,
                                carry=jnp.zeros((16,), jnp.int32))
            def _(l, lvec):
                vals = plsc.load_gather(stgs[b], [ivec, svec, lvec])
                asm[q0 + l, pl.ds(col, 16)] = vals
                return lvec + 1
            return 0
        lax.fori_loop(0, 8, souter, 0)

    def start_out(g):
        pltpu.async_copy(asm, out_hbm.at[:, pl.ds(base + g * BLK, BLK)],
                         osem)

    def wait_out(g):
        pltpu.make_async_copy(asm,
                              out_hbm.at[:, pl.ds(base + g * BLK, BLK)],
                              osem).wait()

    fire(0, 0)

    def pair(j, _):
        c0 = 2 * j
        fire(c0 + 1, 1)
        drain(0)

        @pl.when(jnp.logical_and(j % 4 == 0, j >= 4))
        def _():
            wait_out(j // 4 - 1)

        transpose(0, (c0 % 8) * CH)

        @pl.when(j < NCHUNK // 2 - 1)
        def _():
            fire(c0 + 2, 0)

        drain(1)
        transpose(1, ((c0 + 1) % 8) * CH)

        @pl.when(j % 4 == 3)
        def _():
            start_out(j // 4)

        return 0

    lax.fori_loop(0, NCHUNK // 2, pair, 0)
    wait_out(NBLK - 1)


_gather = functools.partial(
    pl.kernel,
    mesh=plsc.VectorSubcoreMesh(core_axis_name="c", subcore_axis_name="s"),
    out_type=jax.ShapeDtypeStruct((D, B), jnp.float32),
    scratch_types=[
        pltpu.VMEM((BPW,), jnp.int32),
        pltpu.VMEM((CH, 8, 64), jnp.float32),
        pltpu.VMEM((CH, 8, 64), jnp.float32),
        pltpu.VMEM((D, BLK), jnp.float32),
        pltpu.SemaphoreType.DMA,
        pltpu.SemaphoreType.DMA,
        pltpu.SemaphoreType.DMA,
    ],
    compiler_params=pltpu.CompilerParams(use_tc_tiling_on_sc=True,
                                         needs_layout_passes=False),
)(_gbody)


def kernel(x, params):
    lin = _pack(x)
    table = params.reshape(V, 8, 64)
    out2d = _gather(table, lin)
    return out2d.reshape(8, 64, B).transpose(2, 0, 1)


# speedup vs baseline: 1.6884x; 1.0007x over previous
import functools

import jax
import jax.numpy as jnp
from jax import lax
from jax.experimental import pallas as pl
from jax.experimental.pallas import tpu as pltpu
from jax.experimental.pallas import tpu_sc as plsc

B = 16384
V = 32768
NB = 15
NC = 2
NS = 16
NW = NC * NS
BPW = B // NW
CH = 16
NCHUNK = BPW // CH
BLK = 128
NBLK = BPW // BLK
D = 512


def _pack_body(x_ref, lin_ref):
    i = lax.broadcasted_iota(jnp.int32, (NB, 1), 0)
    w = jnp.left_shift(jnp.int32(1), NB - 1 - i)
    lin_ref[...] = jnp.sum(x_ref[...] * w, axis=0)


_pack = pl.pallas_call(
    _pack_body,
    out_shape=jax.ShapeDtypeStruct((B,), jnp.int32),
)


def _gbody(params_hbm, lin_hbm, out_hbm, idx_v, stg0, stg1, stg2, asm,
           gsem0, gsem1, gsem2, osem):
    wid = lax.axis_index("s") * NC + lax.axis_index("c")
    base = wid * BPW

    pltpu.sync_copy(lin_hbm.at[pl.ds(base, BPW)], idx_v)

    stgs = (stg0, stg1, stg2)
    gsems = (gsem0, gsem1, gsem2)
    ivec = lax.broadcasted_iota(jnp.int32, (16,), 0)

    def fire(ci, b):
        v = idx_v[pl.ds(ci * CH, CH)]
        for l in range(CH):
            pltpu.async_copy(params_hbm.at[v[l]], stgs[b].at[l], gsems[b])

    def drain(b):
        pltpu.make_async_copy(params_hbm.at[pl.ds(0, CH)], stgs[b],
                              gsems[b]).wait()

    def transpose(b, col):
        def souter(s, _):
            svec = jnp.full((16,), s, jnp.int32)
            q0 = s * 64

            @plsc.parallel_loop(0, 64, step=1, unroll=16,
                                carry=jnp.zeros((16,), jnp.int32))
            def _(l, lvec):
                vals = plsc.load_gather(stgs[b], [ivec, svec, lvec])
                asm[q0 + l, pl.ds(col, 16)] = vals
                return lvec + 1
            return 0
        lax.fori_loop(0, 8, souter, 0)

    def start_out(g):
        pltpu.async_copy(asm, out_hbm.at[:, pl.ds(base + g * BLK, BLK)],
                         osem)

    def wait_out(g):
        pltpu.make_async_copy(asm,
                              out_hbm.at[:, pl.ds(base + g * BLK, BLK)],
                              osem).wait()

    fire(0, 0)
    fire(1, 1)

    def step(c, _):
        blk = c // 8
        for b in range(3):
            @pl.when(c % 3 == b)
            def _():
                drain(b)

                @pl.when(jnp.logical_and(c % 8 == 0, c >= 8))
                def _():
                    wait_out(blk - 1)

                @pl.when(c + 2 < NCHUNK)
                def _():
                    fire(c + 2, (b + 2) % 3)

                transpose(b, (c % 8) * CH)

                @pl.when(c % 8 == 7)
                def _():
                    start_out(blk)
        return 0

    lax.fori_loop(0, NCHUNK, step, 0)
    wait_out(NBLK - 1)


_gather = functools.partial(
    pl.kernel,
    mesh=plsc.VectorSubcoreMesh(core_axis_name="c", subcore_axis_name="s"),
    out_type=jax.ShapeDtypeStruct((D, B), jnp.float32),
    scratch_types=[
        pltpu.VMEM((BPW,), jnp.int32),
        pltpu.VMEM((CH, 8, 64), jnp.float32),
        pltpu.VMEM((CH, 8, 64), jnp.float32),
        pltpu.VMEM((CH, 8, 64), jnp.float32),
        pltpu.VMEM((D, BLK), jnp.float32),
        pltpu.SemaphoreType.DMA,
        pltpu.SemaphoreType.DMA,
        pltpu.SemaphoreType.DMA,
        pltpu.SemaphoreType.DMA,
    ],
    compiler_params=pltpu.CompilerParams(use_tc_tiling_on_sc=True,
                                         needs_layout_passes=False),
)(_gbody)


def kernel(x, params):
    lin = _pack(x)
    table = params.reshape(V, 8, 64)
    out2d = _gather(table, lin)
    return out2d.reshape(8, 64, B).transpose(2, 0, 1)
